# Initial kernel scaffold; baseline (speedup 1.0000x reference)
import jax
import jax.numpy as jnp
from jax.experimental import pallas as pl


def kernel(src_seq, structure_emb):
    raise NotImplementedError("write your pallas kernel here")


# speedup vs baseline: 6.2769x; 6.2769x over previous
import functools

import jax
import jax.numpy as jnp
from jax import lax
from jax.experimental import pallas as pl
from jax.experimental.pallas import tpu as pltpu
from jax.experimental.pallas import tpu_sc as plsc

_B = 8
_H = 16
_P = 64
_D = 64
_NPOS = 6
_ROWS_PER_J = _P * _P
_NC = 2
_NS = 16
_NW = _NC * _NS
_QUARTERS = _NW // _B
_CHUNK = _ROWS_PER_J // _QUARTERS
_PAIRS = _CHUNK // 2
_GATHER = 128
_NGATHER = _PAIRS // _GATHER
_L = 16
_HALF = _B * _ROWS_PER_J // 2

_mesh = plsc.VectorSubcoreMesh(core_axis_name="c", subcore_axis_name="s")


@functools.partial(
    pl.kernel,
    mesh=_mesh,
    out_type=jax.ShapeDtypeStruct((_B * _H * _ROWS_PER_J // 2, 2 * _D), jnp.float32),
    scratch_types=[
        pltpu.VMEM((_PAIRS,), jnp.int32),
        pltpu.VMEM((_PAIRS,), jnp.int32),
        pltpu.VMEM((_NGATHER, _GATHER), jnp.int32),
        pltpu.VMEM((_PAIRS, 2 * _D), jnp.float32),
        pltpu.SemaphoreType.DMA,
        pltpu.SemaphoreType.DMA,
    ],
)
def _encode(idx_hbm, pt_hbm, out_hbm, ev_v, od_v, pidx_v, rows_v, gsem, wsem):
    wid = lax.axis_index("s") * _NC + lax.axis_index("c")
    j = wid % _B
    q = wid // _B

    base = pl.multiple_of(j * (_ROWS_PER_J // 2) + q * _PAIRS, _PAIRS)
    pltpu.sync_copy(idx_hbm.at[pl.ds(base, _PAIRS)], ev_v)
    pltpu.sync_copy(idx_hbm.at[pl.ds(base + _HALF, _PAIRS)], od_v)

    for g in range(_PAIRS // _L):
        ev = ev_v[pl.ds(g * _L, _L)]
        od = od_v[pl.ds(g * _L, _L)]
        pidx_v[g // 8, pl.ds((g % 8) * _L, _L)] = ev * _NPOS + od

    gathers = [
        pltpu.async_copy(
            pt_hbm.at[pidx_v.at[i]],
            rows_v.at[pl.ds(i * _GATHER, _GATHER)],
            gsem,
        )
        for i in range(_NGATHER)
    ]
    for g in gathers:
        g.wait()

    writes = []
    for b in range(_B):
        for dh in (0, _B):
            start = pl.multiple_of(
                ((b * _H + j + dh) * _ROWS_PER_J + q * _CHUNK) // 2, _PAIRS
            )
            writes.append(
                pltpu.async_copy(rows_v, out_hbm.at[pl.ds(start, _PAIRS)], wsem)
            )
    for w in writes:
        w.wait()


def kernel(src_seq, structure_emb):
    flat = src_seq.reshape(_B * _ROWS_PER_J).astype(jnp.int32)
    pairs = flat.reshape(-1, 2)
    idx_split = jnp.concatenate([pairs[:, 0], pairs[:, 1]])
    emb = structure_emb.astype(jnp.float32)
    pair_table = jnp.concatenate(
        [jnp.repeat(emb, _NPOS, axis=0), jnp.tile(emb, (_NPOS, 1))], axis=1
    )
    out_flat = _encode(idx_split, pair_table)
    return out_flat.reshape(_B, _H, _P, _P, _D)


# speedup vs baseline: 11.2933x; 1.7992x over previous
import functools

import jax
import jax.numpy as jnp
from jax import lax
from jax.experimental import pallas as pl
from jax.experimental.pallas import tpu as pltpu
from jax.experimental.pallas import tpu_sc as plsc

_B = 8
_H = 16
_P = 64
_D = 64
_NPOS = 6
_ROWS_PER_J = _P * _P
_NC = 2
_NS = 16
_NW = _NC * _NS
_QUARTERS = _NW // _B
_CHUNK = _ROWS_PER_J // _QUARTERS
_PAIRS = _CHUNK // 2
_GATHER = 128
_NGATHER = _PAIRS // _GATHER
_L = 16
_HALF = _B * _ROWS_PER_J // 2

_mesh = plsc.VectorSubcoreMesh(core_axis_name="c", subcore_axis_name="s")


@functools.partial(
    pl.kernel,
    mesh=_mesh,
    out_type=jax.ShapeDtypeStruct((_B * _ROWS_PER_J // 2, 2 * _D), jnp.float32),
    scratch_types=[
        pltpu.VMEM((_PAIRS,), jnp.int32),
        pltpu.VMEM((_PAIRS,), jnp.int32),
        pltpu.VMEM((_NGATHER, _GATHER), jnp.int32),
        pltpu.VMEM((_PAIRS, 2 * _D), jnp.float32),
        pltpu.SemaphoreType.DMA,
    ],
)
def _encode(idx_hbm, pt_hbm, g_hbm, ev_v, od_v, pidx_v, rows_v, gsem):
    wid = lax.axis_index("s") * _NC + lax.axis_index("c")
    j = wid % _B
    q = wid // _B

    base = pl.multiple_of(j * (_ROWS_PER_J // 2) + q * _PAIRS, _PAIRS)
    pltpu.sync_copy(idx_hbm.at[pl.ds(base, _PAIRS)], ev_v)
    pltpu.sync_copy(idx_hbm.at[pl.ds(base + _HALF, _PAIRS)], od_v)

    for g in range(_PAIRS // _L):
        ev = ev_v[pl.ds(g * _L, _L)]
        od = od_v[pl.ds(g * _L, _L)]
        pidx_v[g // 8, pl.ds((g % 8) * _L, _L)] = ev * _NPOS + od

    gathers = [
        pltpu.async_copy(
            pt_hbm.at[pidx_v.at[i]],
            rows_v.at[pl.ds(i * _GATHER, _GATHER)],
            gsem,
        )
        for i in range(_NGATHER)
    ]
    for g in gathers:
        g.wait()

    pltpu.sync_copy(rows_v, g_hbm.at[pl.ds(base, _PAIRS)])


_PC = 4


def _bcast_body(g_ref, out_ref):
    g = g_ref[...]
    g4 = g.reshape(_B, _PC, _P, _D)
    out6 = jnp.broadcast_to(g4[None, None], (_B, 2, _B, _PC, _P, _D))
    out_ref[...] = out6.reshape(_B, _H, _PC, _P, _D)


_broadcast = pl.pallas_call(
    _bcast_body,
    grid=(_P // _PC,),
    in_specs=[pl.BlockSpec((_B, _PC * _P, _D), lambda c: (0, c, 0))],
    out_specs=pl.BlockSpec(
        (_B, _H, _PC, _P, _D), lambda c: (0, 0, c, 0, 0)
    ),
    out_shape=jax.ShapeDtypeStruct((_B, _H, _P, _P, _D), jnp.float32),
)


def kernel(src_seq, structure_emb):
    flat = src_seq.reshape(_B * _ROWS_PER_J).astype(jnp.int32)
    pairs = flat.reshape(-1, 2)
    idx_split = jnp.concatenate([pairs[:, 0], pairs[:, 1]])
    emb = structure_emb.astype(jnp.float32)
    pair_table = jnp.concatenate(
        [jnp.repeat(emb, _NPOS, axis=0), jnp.tile(emb, (_NPOS, 1))], axis=1
    )
    g_pairs = _encode(idx_split, pair_table)
    g = g_pairs.reshape(_B, _ROWS_PER_J, _D)
    return _broadcast(g)
